# baseline (speedup 1.0000x reference)
import functools

import jax
import jax.numpy as jnp
from jax import lax
from jax.experimental import pallas as pl
from jax.experimental.pallas import tpu as pltpu
from jax.experimental.pallas import tpu_sc as plsc

N = 10000
E = 320000
D_IN = 128
D_HID = 16
N_CLS = 40

NC = 2
NS = 16
NW = NC * NS
CH = 128
NCH = E // CH
KT = NCH // NW
KREM = NCH - KT * NW
KD = NCH // NS
KDREM = NCH - KD * NS
NPAD = 10240
ROWS = NPAD // NS
NB = 8
NGRP = KT // NB
NTAIL = KT - NGRP * NB
DGRP = KD // NB
DTAIL = KD - DGRP * NB
NFULL = N // ROWS
NLAST = N - NFULL * ROWS

_MESH = plsc.VectorSubcoreMesh(core_axis_name="c", subcore_axis_name="s")
_SC_PARAMS = pltpu.CompilerParams(use_tc_tiling_on_sc=False)
_SC_PARAMS_NL = pltpu.CompilerParams(use_tc_tiling_on_sc=False,
                                     needs_layout_passes=False)


def _newton_rsqrt(d):
    i = plsc.bitcast(d, jnp.int32)
    y = plsc.bitcast(jnp.int32(0x5F3759DF) - (i >> 1), jnp.float32)
    for _ in range(3):
        y = y * (1.5 - 0.5 * d * y * y)
    return y


@functools.partial(
    pl.kernel,
    out_type=[
        jax.ShapeDtypeStruct((NC, NPAD, D_HID), jnp.float32),
        jax.ShapeDtypeStruct((NC * NPAD,), jnp.float32),
    ],
    mesh=_MESH,
    compiler_params=_SC_PARAMS_NL,
    scratch_types=[
        pltpu.VMEM((KD, CH), jnp.int32),
        pltpu.VMEM((KT, CH), jnp.int32),
        pltpu.VMEM((KT, CH), jnp.int32),
        pltpu.VMEM((1, CH), jnp.int32),
        pltpu.VMEM((1, CH), jnp.int32),
        pltpu.VMEM((1, CH), jnp.int32),
        pltpu.VMEM((CH,), jnp.float32),
        pltpu.VMEM((ROWS,), jnp.float32),
        pltpu.VMEM((ROWS, D_HID), jnp.float32),
        pltpu.VMEM_SHARED((NPAD,), jnp.float32),
        pltpu.VMEM_SHARED((NPAD, D_HID), jnp.float32),
        pltpu.VMEM_SHARED((N, D_HID), jnp.float32),
    ] + [pltpu.VMEM((CH, D_HID), jnp.float32)] * NB
      + [pltpu.SemaphoreType.DMA] * (2 * NB),
)
def _deg_agg16(edge_hbm, h1_hbm, zeros1_hbm, zeros_hbm, ones_hbm, out_hbm,
               inv_hbm, dstd_v, src_v, dst_v, xd_v, xs_v, xdst_v, ones_v,
               inv_v, h1_v, deg_sh, agg_sh, hs_sh, *bufs_sems):
    rb = bufs_sems[:NB]
    gsem = bufs_sems[NB:2 * NB]
    ssem = bufs_sems[2 * NB:]
    cid = lax.axis_index("c")
    sid = lax.axis_index("s")
    g = sid * NC + cid

    pltpu.sync_copy(zeros1_hbm.at[pl.ds(sid * ROWS, ROWS)],
                    deg_sh.at[pl.ds(sid * ROWS, ROWS)])
    pltpu.sync_copy(zeros_hbm.at[pl.ds(sid * ROWS, ROWS)],
                    agg_sh.at[pl.ds(sid * ROWS, ROWS)])
    pltpu.sync_copy(ones_hbm, ones_v)
    pltpu.sync_copy(edge_hbm.at[1, pl.ds(sid * KD, KD)], dstd_v)
    pltpu.sync_copy(edge_hbm.at[0, pl.ds(g * KT, KT)], src_v)
    pltpu.sync_copy(edge_hbm.at[1, pl.ds(g * KT, KT)], dst_v)

    @pl.when(sid < KDREM)
    def _():
        pltpu.sync_copy(edge_hbm.at[1, pl.ds(NS * KD + sid, 1)], xd_v)

    @pl.when(g < KREM)
    def _():
        pltpu.sync_copy(edge_hbm.at[0, pl.ds(NW * KT + g, 1)], xs_v)
        pltpu.sync_copy(edge_hbm.at[1, pl.ds(NW * KT + g, 1)], xdst_v)

    plsc.subcore_barrier()

    def deg_body(gi, carry):
        base = gi * NB
        cps = [pltpu.async_copy(ones_v, deg_sh.at[dstd_v.at[base + b]],
                                gsem[b], add=True)
               for b in range(NB)]
        for cp in cps:
            cp.wait()
        return carry

    lax.fori_loop(0, DGRP, deg_body, 0)
    tcps = [pltpu.async_copy(ones_v, deg_sh.at[dstd_v.at[DGRP * NB + b]],
                             gsem[b], add=True)
            for b in range(DTAIL)]
    for cp in tcps:
        cp.wait()

    @pl.when(sid < KDREM)
    def _():
        pltpu.async_copy(ones_v, deg_sh.at[xd_v.at[0]],
                         gsem[0], add=True).wait()

    plsc.subcore_barrier()

    pltpu.sync_copy(deg_sh.at[pl.ds(sid * ROWS, ROWS)], inv_v)

    def inv_body(r, carry):
        d = inv_v[pl.ds(r * 16, 16)] + 1.0
        inv_v[pl.ds(r * 16, 16)] = _newton_rsqrt(d)
        return carry

    lax.fori_loop(0, ROWS // 16, inv_body, 0)
    pltpu.sync_copy(inv_v, inv_hbm.at[pl.ds(cid * NPAD + sid * ROWS, ROWS)])

    def scale_rows(nrows):
        pltpu.sync_copy(h1_hbm.at[pl.ds(sid * ROWS, nrows)],
                        h1_v.at[pl.ds(0, nrows)])

        def scale_body(r, carry):
            bc = plsc.load_gather(inv_v, [jnp.full((16,), r, jnp.int32)])
            h1_v[r, :] = h1_v[r, :] * bc
            return carry

        lax.fori_loop(0, nrows, scale_body, 0)
        pltpu.sync_copy(h1_v.at[pl.ds(0, nrows)],
                        hs_sh.at[pl.ds(sid * ROWS, nrows)])

    @pl.when(sid < NFULL)
    def _():
        scale_rows(ROWS)

    @pl.when(sid == NFULL)
    def _():
        scale_rows(NLAST)

    plsc.subcore_barrier()

    def body(gi, carry):
        base = gi * NB
        gcps = [pltpu.async_copy(hs_sh.at[src_v.at[base + b]], rb[b], gsem[b])
                for b in range(NB)]
        scps = []
        for b in range(NB):
            gcps[b].wait()
            scps.append(pltpu.async_copy(
                rb[b], agg_sh.at[dst_v.at[base + b]], ssem[b], add=True))
        for cp in scps:
            cp.wait()
        return carry

    lax.fori_loop(0, NGRP, body, 0)
    base = NGRP * NB
    gcps = [pltpu.async_copy(hs_sh.at[src_v.at[base + b]], rb[b], gsem[b])
            for b in range(NTAIL)]
    scps = []
    for b in range(NTAIL):
        gcps[b].wait()
        scps.append(pltpu.async_copy(
            rb[b], agg_sh.at[dst_v.at[base + b]], ssem[b], add=True))
    for cp in scps:
        cp.wait()

    @pl.when(g < KREM)
    def _():
        pltpu.async_copy(hs_sh.at[xs_v.at[0]], rb[0], gsem[0]).wait()
        pltpu.async_copy(rb[0], agg_sh.at[xdst_v.at[0]],
                         ssem[0], add=True).wait()

    plsc.subcore_barrier()
    pltpu.sync_copy(agg_sh.at[pl.ds(sid * ROWS, ROWS)],
                    out_hbm.at[cid, pl.ds(sid * ROWS, ROWS)])


@functools.partial(
    pl.kernel,
    out_type=jax.ShapeDtypeStruct((NC, NPAD, N_CLS), jnp.float32),
    mesh=_MESH,
    compiler_params=_SC_PARAMS,
    scratch_types=[
        pltpu.VMEM((KT, CH), jnp.int32),
        pltpu.VMEM((KT, CH), jnp.int32),
        pltpu.VMEM((1, CH), jnp.int32),
        pltpu.VMEM((1, CH), jnp.int32),
        pltpu.VMEM_SHARED((NPAD, N_CLS), jnp.float32),
        pltpu.VMEM_SHARED((N, N_CLS), jnp.float32),
    ] + [pltpu.VMEM((CH, N_CLS), jnp.float32)] * NB
      + [pltpu.SemaphoreType.DMA] * (2 * NB),
)
def _agg40(edge_hbm, hs_hbm, zeros_hbm, out_hbm,
           src_v, dst_v, xs_v, xdst_v, agg_sh, hs_sh, *bufs_sems):
    rb = bufs_sems[:NB]
    gsem = bufs_sems[NB:2 * NB]
    ssem = bufs_sems[2 * NB:]
    cid = lax.axis_index("c")
    sid = lax.axis_index("s")
    g = sid * NC + cid
    pltpu.sync_copy(zeros_hbm.at[pl.ds(sid * ROWS, ROWS)],
                    agg_sh.at[pl.ds(sid * ROWS, ROWS)])
    pltpu.sync_copy(edge_hbm.at[0, pl.ds(g * KT, KT)], src_v)
    pltpu.sync_copy(edge_hbm.at[1, pl.ds(g * KT, KT)], dst_v)

    @pl.when(g < KREM)
    def _():
        pltpu.sync_copy(edge_hbm.at[0, pl.ds(NW * KT + g, 1)], xs_v)
        pltpu.sync_copy(edge_hbm.at[1, pl.ds(NW * KT + g, 1)], xdst_v)

    pltpu.sync_copy(hs_hbm.at[pl.ds(sid * (N // NS), N // NS)],
                    hs_sh.at[pl.ds(sid * (N // NS), N // NS)])
    plsc.subcore_barrier()

    def body(gi, carry):
        base = gi * NB
        gcps = [pltpu.async_copy(hs_sh.at[src_v.at[base + b]], rb[b], gsem[b])
                for b in range(NB)]
        scps = []
        for b in range(NB):
            gcps[b].wait()
            scps.append(pltpu.async_copy(
                rb[b], agg_sh.at[dst_v.at[base + b]], ssem[b], add=True))
        for cp in scps:
            cp.wait()
        return carry

    lax.fori_loop(0, NGRP, body, 0)
    base = NGRP * NB
    gcps = [pltpu.async_copy(hs_sh.at[src_v.at[base + b]], rb[b], gsem[b])
            for b in range(NTAIL)]
    scps = []
    for b in range(NTAIL):
        gcps[b].wait()
        scps.append(pltpu.async_copy(
            rb[b], agg_sh.at[dst_v.at[base + b]], ssem[b], add=True))
    for cp in scps:
        cp.wait()

    @pl.when(g < KREM)
    def _():
        pltpu.async_copy(hs_sh.at[xs_v.at[0]], rb[0], gsem[0]).wait()
        pltpu.async_copy(rb[0], agg_sh.at[xdst_v.at[0]],
                         ssem[0], add=True).wait()

    plsc.subcore_barrier()
    pltpu.sync_copy(agg_sh.at[pl.ds(sid * ROWS, ROWS)],
                    out_hbm.at[cid, pl.ds(sid * ROWS, ROWS)])


_GRID_R = 2000
_GRID = N // _GRID_R


def _h1_body(x_ref, w_ref, h1_ref):
    h1_ref[...] = jnp.dot(x_ref[...], w_ref[...],
                          preferred_element_type=jnp.float32)


def _h1(x, w0):
    return pl.pallas_call(
        _h1_body,
        grid=(_GRID,),
        in_specs=[
            pl.BlockSpec((_GRID_R, D_IN), lambda i: (i, 0)),
            pl.BlockSpec((D_IN, D_HID), lambda i: (0, 0)),
        ],
        out_specs=pl.BlockSpec((_GRID_R, D_HID), lambda i: (i, 0)),
        out_shape=jax.ShapeDtypeStruct((N, D_HID), jnp.float32),
    )(x, w0)


def _layer1_hs2_body(p_ref, h1_ref, inv_ref, b0_ref, w1_ref, hs2_ref):
    inv = inv_ref[0]
    agg = p_ref[0] + p_ref[1] + h1_ref[...] * inv
    out1 = jnp.maximum(agg * inv + b0_ref[...], 0.0)
    hs2_ref[...] = jnp.dot(out1, w1_ref[...],
                           preferred_element_type=jnp.float32) * inv


def _layer1_hs2(p1, h1, inv, b0r, w1):
    return pl.pallas_call(
        _layer1_hs2_body,
        grid=(_GRID,),
        in_specs=[
            pl.BlockSpec((NC, _GRID_R, D_HID), lambda i: (0, i, 0)),
            pl.BlockSpec((_GRID_R, D_HID), lambda i: (i, 0)),
            pl.BlockSpec((1, _GRID_R, 1), lambda i: (0, i, 0)),
            pl.BlockSpec((1, D_HID), lambda i: (0, 0)),
            pl.BlockSpec((D_HID, N_CLS), lambda i: (0, 0)),
        ],
        out_specs=pl.BlockSpec((_GRID_R, N_CLS), lambda i: (i, 0)),
        out_shape=jax.ShapeDtypeStruct((N, N_CLS), jnp.float32),
    )(p1, h1, inv, b0r, w1)


def _layer2_out_body(p_ref, hs2_ref, inv_ref, b1_ref, out_ref):
    agg = p_ref[0] + p_ref[1] + hs2_ref[...]
    out_ref[...] = agg * inv_ref[0] + b1_ref[...]


def _layer2_out(p2, hs2, inv, b1r):
    return pl.pallas_call(
        _layer2_out_body,
        grid=(_GRID,),
        in_specs=[
            pl.BlockSpec((NC, _GRID_R, N_CLS), lambda i: (0, i, 0)),
            pl.BlockSpec((_GRID_R, N_CLS), lambda i: (i, 0)),
            pl.BlockSpec((1, _GRID_R, 1), lambda i: (0, i, 0)),
            pl.BlockSpec((1, N_CLS), lambda i: (0, 0)),
        ],
        out_specs=pl.BlockSpec((_GRID_R, N_CLS), lambda i: (i, 0)),
        out_shape=jax.ShapeDtypeStruct((N, N_CLS), jnp.float32),
    )(p2, hs2, inv, b1r)


def kernel(x, edge_index, W0, b0, W1, b1):
    edge_t = edge_index.astype(jnp.int32).reshape(2, NCH, CH)

    z1 = jnp.zeros((NPAD,), jnp.float32)
    z16 = jnp.zeros((NPAD, D_HID), jnp.float32)
    z40 = jnp.zeros((NPAD, N_CLS), jnp.float32)
    ones = jnp.ones((CH,), jnp.float32)

    h1 = _h1(x, W0)
    p1, inv_flat = _deg_agg16(edge_t, h1, z1, z16, ones)
    inv = inv_flat.reshape(NC, NPAD, 1)

    hs2 = _layer1_hs2(p1, h1, inv, b0.reshape(1, D_HID), W1)

    p2 = _agg40(edge_t, hs2, z40)
    return _layer2_out(p2, hs2, inv, b1.reshape(1, N_CLS))


# speedup vs baseline: 1.0090x; 1.0090x over previous
import functools

import jax
import jax.numpy as jnp
from jax import lax
from jax.experimental import pallas as pl
from jax.experimental.pallas import tpu as pltpu
from jax.experimental.pallas import tpu_sc as plsc

N = 10000
E = 320000
D_IN = 128
D_HID = 16
N_CLS = 40

NC = 2
NS = 16
NW = NC * NS
CH = 128
NCH = E // CH
KT = NCH // NW
KREM = NCH - KT * NW
NPAD = 10240
ROWS = NPAD // NS
NB = 13
NGRP = KT // NB
NTAIL = KT - NGRP * NB
NB40 = 10
NGRP40 = KT // NB40
NTAIL40 = KT - NGRP40 * NB40

_MESH = plsc.VectorSubcoreMesh(core_axis_name="c", subcore_axis_name="s")
_SC_PARAMS = pltpu.CompilerParams(use_tc_tiling_on_sc=False)


@functools.partial(
    pl.kernel,
    out_type=jax.ShapeDtypeStruct((NC, NPAD), jnp.float32),
    mesh=_MESH,
    compiler_params=_SC_PARAMS,
    scratch_types=[
        pltpu.VMEM((KT, CH), jnp.int32),
        pltpu.VMEM((1, CH), jnp.int32),
        pltpu.VMEM((CH,), jnp.float32),
        pltpu.VMEM_SHARED((NPAD,), jnp.float32),
    ] + [pltpu.SemaphoreType.DMA] * NB,
)
def _deg_kernel(edge_hbm, zeros_hbm, ones_hbm, out_hbm,
                idx_v, xidx_v, ones_v, deg_sh, *sems):
    cid = lax.axis_index("c")
    sid = lax.axis_index("s")
    g = sid * NC + cid
    pltpu.sync_copy(zeros_hbm.at[pl.ds(sid * ROWS, ROWS)],
                    deg_sh.at[pl.ds(sid * ROWS, ROWS)])
    pltpu.sync_copy(ones_hbm, ones_v)
    pltpu.sync_copy(edge_hbm.at[1, pl.ds(g * KT, KT)], idx_v)

    @pl.when(g < KREM)
    def _():
        pltpu.sync_copy(edge_hbm.at[1, pl.ds(NW * KT + g, 1)], xidx_v)

    plsc.subcore_barrier()

    def body(gi, carry):
        base = gi * NB
        cps = [pltpu.async_copy(ones_v, deg_sh.at[idx_v.at[base + b]],
                                sems[b], add=True)
               for b in range(NB)]
        for cp in cps:
            cp.wait()
        return carry

    lax.fori_loop(0, NGRP, body, 0)
    tcps = [pltpu.async_copy(ones_v, deg_sh.at[idx_v.at[NGRP * NB + b]],
                             sems[b], add=True)
            for b in range(NTAIL)]
    for cp in tcps:
        cp.wait()

    @pl.when(g < KREM)
    def _():
        pltpu.async_copy(ones_v, deg_sh.at[xidx_v.at[0]],
                         sems[0], add=True).wait()

    plsc.subcore_barrier()
    pltpu.sync_copy(deg_sh.at[pl.ds(sid * ROWS, ROWS)],
                    out_hbm.at[cid, pl.ds(sid * ROWS, ROWS)])


def _make_agg_kernel(D, NB, NGRP, NTAIL):
    @functools.partial(
        pl.kernel,
        out_type=jax.ShapeDtypeStruct((NC, NPAD, D), jnp.float32),
        mesh=_MESH,
        compiler_params=_SC_PARAMS,
        scratch_types=[
            pltpu.VMEM((KT, CH), jnp.int32),
            pltpu.VMEM((KT, CH), jnp.int32),
            pltpu.VMEM((1, CH), jnp.int32),
            pltpu.VMEM((1, CH), jnp.int32),
            pltpu.VMEM_SHARED((NPAD, D), jnp.float32),
            pltpu.VMEM_SHARED((N, D), jnp.float32),
        ] + [pltpu.VMEM((CH, D), jnp.float32)] * NB
          + [pltpu.SemaphoreType.DMA] * (2 * NB),
    )
    def agg(edge_hbm, hs_hbm, zeros_hbm, out_hbm,
            src_v, dst_v, xsrc_v, xdst_v, agg_sh, hs_sh, *bufs_sems):
        rb = bufs_sems[:NB]
        gsem = bufs_sems[NB:2 * NB]
        ssem = bufs_sems[2 * NB:]
        cid = lax.axis_index("c")
        sid = lax.axis_index("s")
        g = sid * NC + cid
        pltpu.sync_copy(zeros_hbm.at[pl.ds(sid * ROWS, ROWS)],
                        agg_sh.at[pl.ds(sid * ROWS, ROWS)])
        pltpu.sync_copy(edge_hbm.at[0, pl.ds(g * KT, KT)], src_v)
        pltpu.sync_copy(edge_hbm.at[1, pl.ds(g * KT, KT)], dst_v)

        @pl.when(g < KREM)
        def _():
            pltpu.sync_copy(edge_hbm.at[0, pl.ds(NW * KT + g, 1)], xsrc_v)
            pltpu.sync_copy(edge_hbm.at[1, pl.ds(NW * KT + g, 1)], xdst_v)

        pltpu.sync_copy(hs_hbm.at[pl.ds(sid * (N // NS), N // NS)],
                        hs_sh.at[pl.ds(sid * (N // NS), N // NS)])
        plsc.subcore_barrier()

        def pair(b, src_row, dst_row):
            cp = pltpu.async_copy(hs_sh.at[src_row], rb[b], gsem[b])
            return cp, dst_row

        def body(gi, carry):
            base = gi * NB
            gcps = [pltpu.async_copy(hs_sh.at[src_v.at[base + b]],
                                     rb[b], gsem[b])
                    for b in range(NB)]
            scps = []
            for b in range(NB):
                gcps[b].wait()
                scps.append(pltpu.async_copy(
                    rb[b], agg_sh.at[dst_v.at[base + b]], ssem[b], add=True))
            for cp in scps:
                cp.wait()
            return carry

        lax.fori_loop(0, NGRP, body, 0)

        base = NGRP * NB
        gcps = [pltpu.async_copy(hs_sh.at[src_v.at[base + b]], rb[b], gsem[b])
                for b in range(NTAIL)]
        scps = []
        for b in range(NTAIL):
            gcps[b].wait()
            scps.append(pltpu.async_copy(
                rb[b], agg_sh.at[dst_v.at[base + b]], ssem[b], add=True))
        for cp in scps:
            cp.wait()

        @pl.when(g < KREM)
        def _():
            pltpu.async_copy(hs_sh.at[xsrc_v.at[0]], rb[0], gsem[0]).wait()
            pltpu.async_copy(rb[0], agg_sh.at[xdst_v.at[0]],
                             ssem[0], add=True).wait()

        plsc.subcore_barrier()
        pltpu.sync_copy(agg_sh.at[pl.ds(sid * ROWS, ROWS)],
                        out_hbm.at[cid, pl.ds(sid * ROWS, ROWS)])

    return agg


_agg16 = _make_agg_kernel(D_HID, NB, NGRP, NTAIL)
_agg40 = _make_agg_kernel(N_CLS, NB40, NGRP40, NTAIL40)


_GRID_R = 2000
_GRID = N // _GRID_R


def _inv_hs1_body(d_ref, x_ref, w_ref, inv_ref, hs_ref):
    deg = d_ref[0] + d_ref[1] + 1.0
    inv = lax.rsqrt(jnp.maximum(deg, 1.0))
    inv_ref[...] = inv
    hs_ref[...] = jnp.dot(x_ref[...], w_ref[...],
                          preferred_element_type=jnp.float32) * inv


def _inv_hs1(degp, x, w0):
    return pl.pallas_call(
        _inv_hs1_body,
        grid=(_GRID,),
        in_specs=[
            pl.BlockSpec((NC, _GRID_R, 1), lambda i: (0, i, 0)),
            pl.BlockSpec((_GRID_R, D_IN), lambda i: (i, 0)),
            pl.BlockSpec((D_IN, D_HID), lambda i: (0, 0)),
        ],
        out_specs=[
            pl.BlockSpec((_GRID_R, 1), lambda i: (i, 0)),
            pl.BlockSpec((_GRID_R, D_HID), lambda i: (i, 0)),
        ],
        out_shape=[
            jax.ShapeDtypeStruct((N, 1), jnp.float32),
            jax.ShapeDtypeStruct((N, D_HID), jnp.float32),
        ],
    )(degp, x, w0)


def _layer1_hs2_body(p_ref, hs1_ref, inv_ref, b0_ref, w1_ref, hs2_ref):
    agg = p_ref[0] + p_ref[1] + hs1_ref[...]
    out1 = jnp.maximum(agg * inv_ref[...] + b0_ref[...], 0.0)
    hs2_ref[...] = jnp.dot(out1, w1_ref[...],
                           preferred_element_type=jnp.float32) * inv_ref[...]


def _layer1_hs2(p1, hs1, inv, b0r, w1):
    return pl.pallas_call(
        _layer1_hs2_body,
        grid=(_GRID,),
        in_specs=[
            pl.BlockSpec((NC, _GRID_R, D_HID), lambda i: (0, i, 0)),
            pl.BlockSpec((_GRID_R, D_HID), lambda i: (i, 0)),
            pl.BlockSpec((_GRID_R, 1), lambda i: (i, 0)),
            pl.BlockSpec((1, D_HID), lambda i: (0, 0)),
            pl.BlockSpec((D_HID, N_CLS), lambda i: (0, 0)),
        ],
        out_specs=pl.BlockSpec((_GRID_R, N_CLS), lambda i: (i, 0)),
        out_shape=jax.ShapeDtypeStruct((N, N_CLS), jnp.float32),
    )(p1, hs1, inv, b0r, w1)


def _layer2_out_body(p_ref, hs2_ref, inv_ref, b1_ref, out_ref):
    agg = p_ref[0] + p_ref[1] + hs2_ref[...]
    out_ref[...] = agg * inv_ref[...] + b1_ref[...]


def _layer2_out(p2, hs2, inv, b1r):
    return pl.pallas_call(
        _layer2_out_body,
        grid=(_GRID,),
        in_specs=[
            pl.BlockSpec((NC, _GRID_R, N_CLS), lambda i: (0, i, 0)),
            pl.BlockSpec((_GRID_R, N_CLS), lambda i: (i, 0)),
            pl.BlockSpec((_GRID_R, 1), lambda i: (i, 0)),
            pl.BlockSpec((1, N_CLS), lambda i: (0, 0)),
        ],
        out_specs=pl.BlockSpec((_GRID_R, N_CLS), lambda i: (i, 0)),
        out_shape=jax.ShapeDtypeStruct((N, N_CLS), jnp.float32),
    )(p2, hs2, inv, b1r)


def kernel(x, edge_index, W0, b0, W1, b1):
    edge_t = edge_index.astype(jnp.int32).reshape(2, NCH, CH)

    z1 = jnp.zeros((NPAD,), jnp.float32)
    z16 = jnp.zeros((NPAD, D_HID), jnp.float32)
    z40 = jnp.zeros((NPAD, N_CLS), jnp.float32)
    ones = jnp.ones((CH,), jnp.float32)

    degp = _deg_kernel(edge_t, z1, ones)
    inv, hs1 = _inv_hs1(degp.reshape(NC, NPAD, 1), x, W0)

    p1 = _agg16(edge_t, hs1, z16)
    hs2 = _layer1_hs2(p1, hs1, inv, b0.reshape(1, D_HID), W1)

    p2 = _agg40(edge_t, hs2, z40)
    return _layer2_out(p2, hs2, inv, b1.reshape(1, N_CLS))


# speedup vs baseline: 1.0146x; 1.0055x over previous
import functools

import jax
import jax.numpy as jnp
from jax import lax
from jax.experimental import pallas as pl
from jax.experimental.pallas import tpu as pltpu
from jax.experimental.pallas import tpu_sc as plsc

N = 10000
E = 320000
D_IN = 128
D_HID = 16
N_CLS = 40

NC = 2
NS = 16
NW = NC * NS
CH = 128
NCH = E // CH
KT = NCH // NW
KREM = NCH - KT * NW
NPAD = 10240
ROWS = NPAD // NS
NB = 8
NGRP = KT // NB
NTAIL = KT - NGRP * NB

_MESH = plsc.VectorSubcoreMesh(core_axis_name="c", subcore_axis_name="s")
_SC_PARAMS = pltpu.CompilerParams(use_tc_tiling_on_sc=False)


@functools.partial(
    pl.kernel,
    out_type=jax.ShapeDtypeStruct((NC, NPAD), jnp.float32),
    mesh=_MESH,
    compiler_params=_SC_PARAMS,
    scratch_types=[
        pltpu.VMEM((KT, CH), jnp.int32),
        pltpu.VMEM((1, CH), jnp.int32),
        pltpu.VMEM((CH,), jnp.float32),
        pltpu.VMEM_SHARED((NPAD,), jnp.float32),
    ] + [pltpu.SemaphoreType.DMA] * NB,
)
def _deg_kernel(edge_hbm, zeros_hbm, ones_hbm, out_hbm,
                idx_v, xidx_v, ones_v, deg_sh, *sems):
    cid = lax.axis_index("c")
    sid = lax.axis_index("s")
    g = sid * NC + cid
    pltpu.sync_copy(zeros_hbm.at[pl.ds(sid * ROWS, ROWS)],
                    deg_sh.at[pl.ds(sid * ROWS, ROWS)])
    pltpu.sync_copy(ones_hbm, ones_v)
    pltpu.sync_copy(edge_hbm.at[1, pl.ds(g * KT, KT)], idx_v)

    @pl.when(g < KREM)
    def _():
        pltpu.sync_copy(edge_hbm.at[1, pl.ds(NW * KT + g, 1)], xidx_v)

    plsc.subcore_barrier()

    def body(gi, carry):
        base = gi * NB
        cps = [pltpu.async_copy(ones_v, deg_sh.at[idx_v.at[base + b]],
                                sems[b], add=True)
               for b in range(NB)]
        for cp in cps:
            cp.wait()
        return carry

    lax.fori_loop(0, NGRP, body, 0)
    tcps = [pltpu.async_copy(ones_v, deg_sh.at[idx_v.at[NGRP * NB + b]],
                             sems[b], add=True)
            for b in range(NTAIL)]
    for cp in tcps:
        cp.wait()

    @pl.when(g < KREM)
    def _():
        pltpu.async_copy(ones_v, deg_sh.at[xidx_v.at[0]],
                         sems[0], add=True).wait()

    plsc.subcore_barrier()
    pltpu.sync_copy(deg_sh.at[pl.ds(sid * ROWS, ROWS)],
                    out_hbm.at[cid, pl.ds(sid * ROWS, ROWS)])


def _make_agg_kernel(D):
    @functools.partial(
        pl.kernel,
        out_type=jax.ShapeDtypeStruct((NC, NPAD, D), jnp.float32),
        mesh=_MESH,
        compiler_params=_SC_PARAMS,
        scratch_types=[
            pltpu.VMEM((KT, CH), jnp.int32),
            pltpu.VMEM((KT, CH), jnp.int32),
            pltpu.VMEM((1, CH), jnp.int32),
            pltpu.VMEM((1, CH), jnp.int32),
            pltpu.VMEM_SHARED((NPAD, D), jnp.float32),
            pltpu.VMEM_SHARED((N, D), jnp.float32),
        ] + [pltpu.VMEM((CH, D), jnp.float32)] * NB
          + [pltpu.SemaphoreType.DMA] * (2 * NB),
    )
    def agg(edge_hbm, hs_hbm, zeros_hbm, out_hbm,
            src_v, dst_v, xsrc_v, xdst_v, agg_sh, hs_sh, *bufs_sems):
        rb = bufs_sems[:NB]
        gsem = bufs_sems[NB:2 * NB]
        ssem = bufs_sems[2 * NB:]
        cid = lax.axis_index("c")
        sid = lax.axis_index("s")
        g = sid * NC + cid
        pltpu.sync_copy(zeros_hbm.at[pl.ds(sid * ROWS, ROWS)],
                        agg_sh.at[pl.ds(sid * ROWS, ROWS)])
        pltpu.sync_copy(edge_hbm.at[0, pl.ds(g * KT, KT)], src_v)
        pltpu.sync_copy(edge_hbm.at[1, pl.ds(g * KT, KT)], dst_v)

        @pl.when(g < KREM)
        def _():
            pltpu.sync_copy(edge_hbm.at[0, pl.ds(NW * KT + g, 1)], xsrc_v)
            pltpu.sync_copy(edge_hbm.at[1, pl.ds(NW * KT + g, 1)], xdst_v)

        pltpu.sync_copy(hs_hbm.at[pl.ds(sid * (N // NS), N // NS)],
                        hs_sh.at[pl.ds(sid * (N // NS), N // NS)])
        plsc.subcore_barrier()

        def pair(b, src_row, dst_row):
            cp = pltpu.async_copy(hs_sh.at[src_row], rb[b], gsem[b])
            return cp, dst_row

        def body(gi, carry):
            base = gi * NB
            gcps = [pltpu.async_copy(hs_sh.at[src_v.at[base + b]],
                                     rb[b], gsem[b])
                    for b in range(NB)]
            scps = []
            for b in range(NB):
                gcps[b].wait()
                scps.append(pltpu.async_copy(
                    rb[b], agg_sh.at[dst_v.at[base + b]], ssem[b], add=True))
            for cp in scps:
                cp.wait()
            return carry

        lax.fori_loop(0, NGRP, body, 0)

        base = NGRP * NB
        gcps = [pltpu.async_copy(hs_sh.at[src_v.at[base + b]], rb[b], gsem[b])
                for b in range(NTAIL)]
        scps = []
        for b in range(NTAIL):
            gcps[b].wait()
            scps.append(pltpu.async_copy(
                rb[b], agg_sh.at[dst_v.at[base + b]], ssem[b], add=True))
        for cp in scps:
            cp.wait()

        @pl.when(g < KREM)
        def _():
            pltpu.async_copy(hs_sh.at[xsrc_v.at[0]], rb[0], gsem[0]).wait()
            pltpu.async_copy(rb[0], agg_sh.at[xdst_v.at[0]],
                             ssem[0], add=True).wait()

        plsc.subcore_barrier()
        pltpu.sync_copy(agg_sh.at[pl.ds(sid * ROWS, ROWS)],
                        out_hbm.at[cid, pl.ds(sid * ROWS, ROWS)])

    return agg


_agg16 = _make_agg_kernel(D_HID)
_agg40 = _make_agg_kernel(N_CLS)


_GRID_R = 2000
_GRID = N // _GRID_R


def _inv_hs1_body(d_ref, x_ref, w_ref, inv_ref, hs_ref):
    deg = d_ref[0] + d_ref[1] + 1.0
    inv = lax.rsqrt(jnp.maximum(deg, 1.0))
    inv_ref[...] = inv
    hs_ref[...] = jnp.dot(x_ref[...], w_ref[...],
                          preferred_element_type=jnp.float32) * inv


def _inv_hs1(degp, x, w0):
    return pl.pallas_call(
        _inv_hs1_body,
        grid=(_GRID,),
        in_specs=[
            pl.BlockSpec((NC, _GRID_R, 1), lambda i: (0, i, 0)),
            pl.BlockSpec((_GRID_R, D_IN), lambda i: (i, 0)),
            pl.BlockSpec((D_IN, D_HID), lambda i: (0, 0)),
        ],
        out_specs=[
            pl.BlockSpec((_GRID_R, 1), lambda i: (i, 0)),
            pl.BlockSpec((_GRID_R, D_HID), lambda i: (i, 0)),
        ],
        out_shape=[
            jax.ShapeDtypeStruct((N, 1), jnp.float32),
            jax.ShapeDtypeStruct((N, D_HID), jnp.float32),
        ],
    )(degp, x, w0)


def _layer1_hs2_body(p_ref, hs1_ref, inv_ref, b0_ref, w1_ref, hs2_ref):
    agg = p_ref[0] + p_ref[1] + hs1_ref[...]
    out1 = jnp.maximum(agg * inv_ref[...] + b0_ref[...], 0.0)
    hs2_ref[...] = jnp.dot(out1, w1_ref[...],
                           preferred_element_type=jnp.float32) * inv_ref[...]


def _layer1_hs2(p1, hs1, inv, b0r, w1):
    return pl.pallas_call(
        _layer1_hs2_body,
        grid=(_GRID,),
        in_specs=[
            pl.BlockSpec((NC, _GRID_R, D_HID), lambda i: (0, i, 0)),
            pl.BlockSpec((_GRID_R, D_HID), lambda i: (i, 0)),
            pl.BlockSpec((_GRID_R, 1), lambda i: (i, 0)),
            pl.BlockSpec((1, D_HID), lambda i: (0, 0)),
            pl.BlockSpec((D_HID, N_CLS), lambda i: (0, 0)),
        ],
        out_specs=pl.BlockSpec((_GRID_R, N_CLS), lambda i: (i, 0)),
        out_shape=jax.ShapeDtypeStruct((N, N_CLS), jnp.float32),
    )(p1, hs1, inv, b0r, w1)


def _layer2_out_body(p_ref, hs2_ref, inv_ref, b1_ref, out_ref):
    agg = p_ref[0] + p_ref[1] + hs2_ref[...]
    out_ref[...] = agg * inv_ref[...] + b1_ref[...]


def _layer2_out(p2, hs2, inv, b1r):
    return pl.pallas_call(
        _layer2_out_body,
        grid=(_GRID,),
        in_specs=[
            pl.BlockSpec((NC, _GRID_R, N_CLS), lambda i: (0, i, 0)),
            pl.BlockSpec((_GRID_R, N_CLS), lambda i: (i, 0)),
            pl.BlockSpec((_GRID_R, 1), lambda i: (i, 0)),
            pl.BlockSpec((1, N_CLS), lambda i: (0, 0)),
        ],
        out_specs=pl.BlockSpec((_GRID_R, N_CLS), lambda i: (i, 0)),
        out_shape=jax.ShapeDtypeStruct((N, N_CLS), jnp.float32),
    )(p2, hs2, inv, b1r)


def kernel(x, edge_index, W0, b0, W1, b1):
    edge_t = edge_index.astype(jnp.int32).reshape(2, NCH, CH)

    z1 = jnp.zeros((NPAD,), jnp.float32)
    z16 = jnp.zeros((NPAD, D_HID), jnp.float32)
    z40 = jnp.zeros((NPAD, N_CLS), jnp.float32)
    ones = jnp.ones((CH,), jnp.float32)

    degp = _deg_kernel(edge_t, z1, ones)
    inv, hs1 = _inv_hs1(degp.reshape(NC, NPAD, 1), x, W0)

    p1 = _agg16(edge_t, hs1, z16)
    hs2 = _layer1_hs2(p1, hs1, inv, b0.reshape(1, D_HID), W1)

    p2 = _agg40(edge_t, hs2, z40)
    return _layer2_out(p2, hs2, inv, b1.reshape(1, N_CLS))
